# baseline (speedup 1.0000x reference)
import functools

import jax
import jax.numpy as jnp
from jax import lax
from jax.experimental import pallas as pl
from jax.experimental.pallas import tpu as pltpu
from jax.experimental.pallas import tpu_sc as plsc

V = 1000000
D = 32
B = 16384
K = 20

NC = 2
NS = 16
NW = NC * NS
NB = B // NW
C = 32
NCHUNK = NB // C


def _sc_body(emb_c, emb_o, cidx4_h, oidx4_h, nidx4_h, cm_h, om_h, nm_h,
             uo_h, ng_h,
             cidx4_v, oidx4_v, nidx4_v, cm_v, om_v, nm_v,
             crows, orows, nrows, uo_v, ng_v, sem):
    i32 = jnp.int32
    wid = lax.axis_index("s") * NC + lax.axis_index("c")
    pltpu.sync_copy(cidx4_h.at[pl.ds(wid * NB, NB)], cidx4_v)
    pltpu.sync_copy(oidx4_h.at[pl.ds(wid * NB, NB)], oidx4_v)
    pltpu.sync_copy(nidx4_h.at[pl.ds(wid * NB * K, NB * K)], nidx4_v)
    pltpu.sync_copy(cm_h.at[pl.ds(wid * NB, NB)], cm_v.at[pl.ds(0, NB)])
    pltpu.sync_copy(om_h.at[pl.ds(wid * NB, NB)], om_v.at[pl.ds(0, NB)])
    pltpu.sync_copy(nm_h.at[pl.ds(wid * NB * K, NB * K)], nm_v.at[pl.ds(0, NB * K)])

    iota = lax.broadcasted_iota(i32, (16,), 0)
    zeros = jnp.zeros((16,), i32)

    def splat(vec, lane_const):
        return jnp.take_along_axis(vec, lane_const, axis=0,
                                   mode="promise_in_bounds")

    def chunk_body(ch, _):
        cbase = ch * C
        cps = [
            pltpu.async_copy(emb_c.at[cidx4_v.at[pl.ds(cbase, C)]], crows, sem),
            pltpu.async_copy(emb_o.at[oidx4_v.at[pl.ds(cbase, C)]], orows, sem),
        ]
        for j in range(K):
            cps.append(pltpu.async_copy(
                emb_o.at[nidx4_v.at[pl.ds(cbase * K + j * C, C)]],
                nrows.at[pl.ds(j * C, C)], sem))
        for cp in cps:
            cp.wait()

        def bbody(b, _):
            bb = cbase + b
            bvec = jnp.broadcast_to(b, (16,))
            qc = splat(cm_v[pl.ds(bb, 16)], zeros) * 32 + iota
            qo = splat(om_v[pl.ds(bb, 16)], zeros) * 32 + iota
            c0 = plsc.load_gather(crows, [bvec, qc])
            c1 = plsc.load_gather(crows, [bvec, qc + 16])
            o0 = plsc.load_gather(orows, [bvec, qo])
            o1 = plsc.load_gather(orows, [bvec, qo + 16])
            uo_v[pl.ds(b * 16, 16)] = c0 * o0 + c1 * o1
            nb0 = bb * K
            nmv0 = nm_v[pl.ds(nb0, 16)]
            nmv1 = nm_v[pl.ds(nb0 + 16, 16)]
            a0 = jnp.zeros((16,), jnp.float32)
            a1 = jnp.zeros((16,), jnp.float32)
            for k in range(K):
                mv, lane = (nmv0, k) if k < 16 else (nmv1, k - 16)
                qk = splat(mv, jnp.full((16,), lane, i32)) * 32 + iota
                rvec = jnp.broadcast_to(b * K + k, (16,))
                a0 = a0 + plsc.load_gather(nrows, [rvec, qk])
                a1 = a1 + plsc.load_gather(nrows, [rvec, qk + 16])
            ng_v[pl.ds(b * 16, 16)] = c0 * a0 + c1 * a1
            return 0

        lax.fori_loop(0, C, bbody, 0)
        out_base = (wid * NB + cbase) * 16
        pltpu.sync_copy(uo_v, uo_h.at[pl.ds(out_base, C * 16)])
        pltpu.sync_copy(ng_v, ng_h.at[pl.ds(out_base, C * 16)])
        return 0

    lax.fori_loop(0, NCHUNK, chunk_body, 0)


@jax.jit
def _sc_partials(cidx4, oidx4, nidx4, cm, om, nm, emb_c4, emb_o4):
    mesh = plsc.VectorSubcoreMesh(core_axis_name="c", subcore_axis_name="s")
    f32 = jnp.float32
    i32 = jnp.int32
    return pl.kernel(
        _sc_body,
        out_type=(
            jax.ShapeDtypeStruct((B * 16,), f32),
            jax.ShapeDtypeStruct((B * 16,), f32),
        ),
        mesh=mesh,
        compiler_params=pltpu.CompilerParams(use_tc_tiling_on_sc=True,
                                             needs_layout_passes=False),
        scratch_types=[
            pltpu.VMEM((NB,), i32),
            pltpu.VMEM((NB,), i32),
            pltpu.VMEM((NB * K,), i32),
            pltpu.VMEM((NB + 16,), i32),
            pltpu.VMEM((NB + 16,), i32),
            pltpu.VMEM((NB * K + 16,), i32),
            pltpu.VMEM((C, 128), f32),
            pltpu.VMEM((C, 128), f32),
            pltpu.VMEM((K * C, 128), f32),
            pltpu.VMEM((C * 16,), f32),
            pltpu.VMEM((C * 16,), f32),
            pltpu.SemaphoreType.DMA,
        ],
    )(emb_c4, emb_o4, cidx4, oidx4, nidx4, cm, om, nm)


def _tc_body(uo_ref, ng_ref, out_ref):
    uo = uo_ref[...]
    ng = ng_ref[...]
    gi = lax.broadcasted_iota(jnp.int32, (128, 8), 0) // 16
    gj = lax.broadcasted_iota(jnp.int32, (128, 8), 1)
    g = (gi == gj).astype(jnp.float32)
    dn = (((1,), (0,)), ((), ()))
    uos = lax.dot_general(uo, g, dn, preferred_element_type=jnp.float32)
    ngs = lax.dot_general(ng, g, dn, preferred_element_type=jnp.float32)

    def logsig(t):
        return jnp.minimum(t, 0.0) - jnp.log1p(jnp.exp(-jnp.abs(t)))

    loss = logsig(uos) + logsig(-ngs)
    out_ref[0, 0] = -jnp.sum(loss) / jnp.float32(B)


@jax.jit
def _tc_loss(uo2d, ng2d):
    return pl.pallas_call(
        _tc_body,
        out_shape=jax.ShapeDtypeStruct((1, 1), jnp.float32),
        out_specs=pl.BlockSpec(memory_space=pltpu.SMEM),
    )(uo2d, ng2d)


def kernel(center, outside, negative, emb_center, emb_outside):
    cidx = center.reshape(B)
    oidx = outside.reshape(B)
    nidx = negative.reshape(B * K)
    emb_c4 = emb_center.reshape(V // 4, 128)
    emb_o4 = emb_outside.reshape(V // 4, 128)
    uo, ng = _sc_partials(cidx >> 2, oidx >> 2, nidx >> 2,
                          cidx & 3, oidx & 3, nidx & 3,
                          emb_c4, emb_o4)
    out = _tc_loss(uo.reshape(B * 16 // 128, 128), ng.reshape(B * 16 // 128, 128))
    return out[0, 0]
